# baseline (speedup 1.0000x reference)
import functools

import jax
import jax.numpy as jnp
from jax import lax
from jax.experimental import pallas as pl
from jax.experimental.pallas import tpu as pltpu
from jax.experimental.pallas import tpu_sc as plsc

N = 10000
E = 320000
EMB = 128
EDGE_DIM = 16
NUM_FEAT = 128
NUM_CLASSES = 6
L = 5
NGRAPH = 64

NCORE = 2
NSUB = 16
NW = NCORE * NSUB
EDGES_PER_W = E // NW
CHUNK = 80
NCHUNK = EDGES_PER_W // CHUNK
ROWS_PER_TILE = 624
ROWS_REM = N - NSUB * ROWS_PER_TILE
VPR = EMB // 16


def _sc_body(h_hbm, ea_hbm, src_hbm, dst_hbm, zeros_hbm, agg_out,
             shared_agg, src_v, dst_v, h_rows, ea_rows, sem):
    c = lax.axis_index("c")
    s = lax.axis_index("s")
    wid = c * NSUB + s

    pltpu.sync_copy(zeros_hbm.at[pl.ds(s * ROWS_PER_TILE, ROWS_PER_TILE), :],
                    shared_agg.at[pl.ds(s * ROWS_PER_TILE, ROWS_PER_TILE), :])

    @pl.when(s == 0)
    def _zero_rem():
        pltpu.sync_copy(zeros_hbm.at[pl.ds(NSUB * ROWS_PER_TILE, ROWS_REM), :],
                        shared_agg.at[pl.ds(NSUB * ROWS_PER_TILE, ROWS_REM), :])

    plsc.subcore_barrier()

    base_e = wid * EDGES_PER_W

    @pl.loop(0, NCHUNK)
    def _chunk(g):
        b = base_e + g * CHUNK
        pltpu.sync_copy(src_hbm.at[pl.ds(b, CHUNK)], src_v)
        pltpu.sync_copy(dst_hbm.at[pl.ds(b, CHUNK)], dst_v)
        gat = pltpu.async_copy(h_hbm.at[src_v], h_rows, sem)
        pltpu.sync_copy(ea_hbm.at[pl.ds(b, CHUNK), :], ea_rows)
        gat.wait()

        @pl.loop(0, CHUNK, unroll=4)
        def _row(i):
            for j in range(VPR):
                sl = pl.ds(j * 16, 16)
                ea_rows[i, sl] = jnp.maximum(h_rows[i, sl] + ea_rows[i, sl],
                                             0.0)

        pltpu.sync_copy(ea_rows, shared_agg.at[dst_v], add=True)

    plsc.subcore_barrier()
    pltpu.sync_copy(shared_agg.at[pl.ds(s * ROWS_PER_TILE, ROWS_PER_TILE), :],
                    agg_out.at[c, pl.ds(s * ROWS_PER_TILE, ROWS_PER_TILE), :])

    @pl.when(s == 0)
    def _out_rem():
        pltpu.sync_copy(shared_agg.at[pl.ds(NSUB * ROWS_PER_TILE, ROWS_REM), :],
                        agg_out.at[c, pl.ds(NSUB * ROWS_PER_TILE, ROWS_REM), :])


_sc_msg = functools.partial(
    pl.kernel,
    out_type=jax.ShapeDtypeStruct((NCORE, N, EMB), jnp.float32),
    mesh=plsc.VectorSubcoreMesh(core_axis_name="c", subcore_axis_name="s",
                                num_cores=NCORE, num_subcores=NSUB),
    scratch_types=[
        pltpu.VMEM_SHARED((N, EMB), jnp.float32),
        pltpu.VMEM((CHUNK,), jnp.int32),
        pltpu.VMEM((CHUNK,), jnp.int32),
        pltpu.VMEM((CHUNK, EMB), jnp.float32),
        pltpu.VMEM((CHUNK, EMB), jnp.float32),
        pltpu.SemaphoreType.DMA,
    ],
)(_sc_body)


def _dot_def(a, b):
    return jnp.dot(a.astype(jnp.bfloat16), b.astype(jnp.bfloat16),
                   preferred_element_type=jnp.float32)


def _embed_body(x_ref, table_ref, out_ref):
    ids = lax.broadcasted_iota(jnp.int32, (1, NUM_FEAT), 1)
    onehot = (x_ref[:, :] == ids).astype(jnp.float32)
    out_ref[:, :] = jnp.dot(onehot, table_ref[:, :],
                            preferred_element_type=jnp.float32, precision=lax.Precision.HIGHEST)


_embed = pl.pallas_call(
    _embed_body,
    out_shape=jax.ShapeDtypeStruct((N, EMB), jnp.float32),
)


def _ea_body(attr_ref, w_ref, b_ref, out_ref):
    out_ref[:, :] = _dot_def(attr_ref[:, :], w_ref[:, :]) + b_ref[:, :]


_EA_BLK = 4000
_ea_proj = pl.pallas_call(
    _ea_body,
    grid=(E // _EA_BLK,),
    in_specs=[
        pl.BlockSpec((_EA_BLK, EDGE_DIM), lambda i: (i, 0)),
        pl.BlockSpec((EDGE_DIM, EMB), lambda i: (0, 0)),
        pl.BlockSpec((1, EMB), lambda i: (0, 0)),
    ],
    out_specs=pl.BlockSpec((_EA_BLK, EMB), lambda i: (i, 0)),
    out_shape=jax.ShapeDtypeStruct((E, EMB), jnp.float32),
)


def _bn(z, g, b):
    m = jnp.mean(z, axis=0, keepdims=True)
    v = jnp.mean((z - m) ** 2, axis=0, keepdims=True)
    return (z - m) / jnp.sqrt(v + 1e-5) * g + b


def _mlp_body(h_ref, agg_ref, eps_ref, w1_ref, b1_ref, g1_ref, be1_ref,
              w2_ref, b2_ref, g2_ref, be2_ref, out_ref):
    z = (1.0 + eps_ref[0, 0]) * h_ref[:, :] + agg_ref[0] + agg_ref[1]
    z = _dot_def(z, w1_ref[:, :]) + b1_ref[:, :]
    z = jnp.maximum(_bn(z, g1_ref[:, :], be1_ref[:, :]), 0.0)
    z = _dot_def(z, w2_ref[:, :]) + b2_ref[:, :]
    out_ref[:, :] = jnp.maximum(_bn(z, g2_ref[:, :], be2_ref[:, :]), 0.0)


_mlp = pl.pallas_call(
    _mlp_body,
    in_specs=[
        pl.BlockSpec(memory_space=pltpu.VMEM),
        pl.BlockSpec(memory_space=pltpu.VMEM),
        pl.BlockSpec(memory_space=pltpu.SMEM),
        pl.BlockSpec(memory_space=pltpu.VMEM),
        pl.BlockSpec(memory_space=pltpu.VMEM),
        pl.BlockSpec(memory_space=pltpu.VMEM),
        pl.BlockSpec(memory_space=pltpu.VMEM),
        pl.BlockSpec(memory_space=pltpu.VMEM),
        pl.BlockSpec(memory_space=pltpu.VMEM),
        pl.BlockSpec(memory_space=pltpu.VMEM),
        pl.BlockSpec(memory_space=pltpu.VMEM),
    ],
    out_shape=jax.ShapeDtypeStruct((N, EMB), jnp.float32),
)


def _pool_body(h_ref, batch_ref, w1_ref, b1_ref, w2_ref, b2_ref, out_ref):
    gids = lax.broadcasted_iota(jnp.int32, (NGRAPH, 1), 0)
    mask = (batch_ref[:, :] == gids).astype(jnp.float32)
    pooled = jnp.dot(mask, h_ref[:, :], preferred_element_type=jnp.float32, precision=lax.Precision.HIGHEST)
    counts = jnp.sum(mask, axis=1, keepdims=True)
    pooled = pooled / jnp.maximum(counts, 1.0)
    hid = jnp.maximum(_dot_def(pooled, w1_ref[:, :]) + b1_ref[:, :], 0.0)
    out_ref[:, :] = _dot_def(hid, w2_ref[:, :]) + b2_ref[:, :]


_pool_cls = pl.pallas_call(
    _pool_body,
    out_shape=jax.ShapeDtypeStruct((NGRAPH, NUM_CLASSES), jnp.float32),
)


def kernel(x, edge_index, edge_attr, batch, node_table, edge_W, edge_b, eps,
           W1, b1, bn1_g, bn1_b, W2, b2, bn2_g, bn2_b,
           cls_W1, cls_b1, cls_W2, cls_b2):
    src = edge_index[0].astype(jnp.int32)
    dst = edge_index[1].astype(jnp.int32)

    h = _embed(x.astype(jnp.int32).reshape(N, 1), node_table)
    ea = _ea_proj(edge_attr, edge_W, edge_b.reshape(1, EMB))
    zeros = jnp.zeros((N, EMB), jnp.float32)

    for i in range(L):
        agg = _sc_msg(h, ea, src, dst, zeros)
        h = _mlp(h, agg, eps[i].reshape(1, 1),
                 W1[i], b1[i].reshape(1, EMB),
                 bn1_g[i].reshape(1, EMB), bn1_b[i].reshape(1, EMB),
                 W2[i], b2[i].reshape(1, EMB),
                 bn2_g[i].reshape(1, EMB), bn2_b[i].reshape(1, EMB))

    return _pool_cls(h, batch.astype(jnp.int32).reshape(1, N),
                     cls_W1, cls_b1.reshape(1, EMB // 2),
                     cls_W2, cls_b2.reshape(1, NUM_CLASSES))


# speedup vs baseline: 1.6209x; 1.6209x over previous
import functools

import jax
import jax.numpy as jnp
from jax import lax
from jax.experimental import pallas as pl
from jax.experimental.pallas import tpu as pltpu
from jax.experimental.pallas import tpu_sc as plsc

N = 10000
E = 320000
EMB = 128
EDGE_DIM = 16
NUM_FEAT = 128
NUM_CLASSES = 6
L = 5
NGRAPH = 64

NCORE = 2
NSUB = 16
NW = NCORE * NSUB
EDGES_PER_W = E // NW
CHUNK = 80
NCHUNK = EDGES_PER_W // CHUNK
ROWS_PER_TILE = 624
ROWS_REM = N - NSUB * ROWS_PER_TILE
VPR = EMB // 16


def _sc_body(h_hbm, ea_hbm, src_hbm, dst_hbm, zeros_hbm, agg_out,
             shared_agg, src_v, dst_v, h_rows, ea_rows, sem):
    c = lax.axis_index("c")
    s = lax.axis_index("s")
    wid = c * NSUB + s

    pltpu.sync_copy(zeros_hbm.at[pl.ds(s * ROWS_PER_TILE, ROWS_PER_TILE), :],
                    shared_agg.at[pl.ds(s * ROWS_PER_TILE, ROWS_PER_TILE), :])

    @pl.when(s == 0)
    def _zero_rem():
        pltpu.sync_copy(zeros_hbm.at[pl.ds(NSUB * ROWS_PER_TILE, ROWS_REM), :],
                        shared_agg.at[pl.ds(NSUB * ROWS_PER_TILE, ROWS_REM), :])

    plsc.subcore_barrier()

    base_e = wid * EDGES_PER_W

    @pl.loop(0, NCHUNK)
    def _chunk(g):
        b = base_e + g * CHUNK
        pltpu.sync_copy(src_hbm.at[pl.ds(b, CHUNK)], src_v)
        pltpu.sync_copy(dst_hbm.at[pl.ds(b, CHUNK)], dst_v)
        gat = pltpu.async_copy(h_hbm.at[src_v], h_rows, sem)
        pltpu.sync_copy(ea_hbm.at[pl.ds(b, CHUNK), :], ea_rows)
        gat.wait()

        @pl.loop(0, CHUNK)
        def _row(i):
            for j in range(VPR):
                sl = pl.ds(j * 16, 16)
                ea_rows[i, sl] = jnp.maximum(h_rows[i, sl] + ea_rows[i, sl],
                                             0.0)

        pltpu.sync_copy(ea_rows, shared_agg.at[dst_v], add=True)

    plsc.subcore_barrier()
    pltpu.sync_copy(shared_agg.at[pl.ds(s * ROWS_PER_TILE, ROWS_PER_TILE), :],
                    agg_out.at[c, pl.ds(s * ROWS_PER_TILE, ROWS_PER_TILE), :])

    @pl.when(s == 0)
    def _out_rem():
        pltpu.sync_copy(shared_agg.at[pl.ds(NSUB * ROWS_PER_TILE, ROWS_REM), :],
                        agg_out.at[c, pl.ds(NSUB * ROWS_PER_TILE, ROWS_REM), :])


_sc_msg = functools.partial(
    pl.kernel,
    out_type=jax.ShapeDtypeStruct((NCORE, N, EMB), jnp.float32),
    mesh=plsc.VectorSubcoreMesh(core_axis_name="c", subcore_axis_name="s",
                                num_cores=NCORE, num_subcores=NSUB),
    scratch_types=[
        pltpu.VMEM_SHARED((N, EMB), jnp.float32),
        pltpu.VMEM((CHUNK,), jnp.int32),
        pltpu.VMEM((CHUNK,), jnp.int32),
        pltpu.VMEM((CHUNK, EMB), jnp.float32),
        pltpu.VMEM((CHUNK, EMB), jnp.float32),
        pltpu.SemaphoreType.DMA,
    ],
)(_sc_body)


def _dot_def(a, b):
    return jnp.dot(a.astype(jnp.bfloat16), b.astype(jnp.bfloat16),
                   preferred_element_type=jnp.float32)


def _embed_body(x_ref, table_ref, out_ref):
    ids = lax.broadcasted_iota(jnp.int32, (1, NUM_FEAT), 1)
    onehot = (x_ref[:, :] == ids).astype(jnp.float32)
    out_ref[:, :] = jnp.dot(onehot, table_ref[:, :],
                            preferred_element_type=jnp.float32, precision=lax.Precision.HIGHEST)


_embed = pl.pallas_call(
    _embed_body,
    out_shape=jax.ShapeDtypeStruct((N, EMB), jnp.float32),
)


def _ea_body(attr_ref, w_ref, b_ref, out_ref):
    out_ref[:, :] = _dot_def(attr_ref[:, :], w_ref[:, :]) + b_ref[:, :]


_EA_BLK = 4000
_ea_proj = pl.pallas_call(
    _ea_body,
    grid=(E // _EA_BLK,),
    in_specs=[
        pl.BlockSpec((_EA_BLK, EDGE_DIM), lambda i: (i, 0)),
        pl.BlockSpec((EDGE_DIM, EMB), lambda i: (0, 0)),
        pl.BlockSpec((1, EMB), lambda i: (0, 0)),
    ],
    out_specs=pl.BlockSpec((_EA_BLK, EMB), lambda i: (i, 0)),
    out_shape=jax.ShapeDtypeStruct((E, EMB), jnp.float32),
)


def _bn(z, g, b):
    m = jnp.mean(z, axis=0, keepdims=True)
    v = jnp.mean((z - m) ** 2, axis=0, keepdims=True)
    return (z - m) / jnp.sqrt(v + 1e-5) * g + b


def _mlp_body(h_ref, agg_ref, eps_ref, w1_ref, b1_ref, g1_ref, be1_ref,
              w2_ref, b2_ref, g2_ref, be2_ref, out_ref):
    z = (1.0 + eps_ref[0, 0]) * h_ref[:, :] + agg_ref[0] + agg_ref[1]
    z = _dot_def(z, w1_ref[:, :]) + b1_ref[:, :]
    z = jnp.maximum(_bn(z, g1_ref[:, :], be1_ref[:, :]), 0.0)
    z = _dot_def(z, w2_ref[:, :]) + b2_ref[:, :]
    out_ref[:, :] = jnp.maximum(_bn(z, g2_ref[:, :], be2_ref[:, :]), 0.0)


_mlp = pl.pallas_call(
    _mlp_body,
    in_specs=[
        pl.BlockSpec(memory_space=pltpu.VMEM),
        pl.BlockSpec(memory_space=pltpu.VMEM),
        pl.BlockSpec(memory_space=pltpu.SMEM),
        pl.BlockSpec(memory_space=pltpu.VMEM),
        pl.BlockSpec(memory_space=pltpu.VMEM),
        pl.BlockSpec(memory_space=pltpu.VMEM),
        pl.BlockSpec(memory_space=pltpu.VMEM),
        pl.BlockSpec(memory_space=pltpu.VMEM),
        pl.BlockSpec(memory_space=pltpu.VMEM),
        pl.BlockSpec(memory_space=pltpu.VMEM),
        pl.BlockSpec(memory_space=pltpu.VMEM),
    ],
    out_shape=jax.ShapeDtypeStruct((N, EMB), jnp.float32),
)


def _pool_body(h_ref, batch_ref, w1_ref, b1_ref, w2_ref, b2_ref, out_ref):
    gids = lax.broadcasted_iota(jnp.int32, (NGRAPH, 1), 0)
    mask = (batch_ref[:, :] == gids).astype(jnp.float32)
    pooled = jnp.dot(mask, h_ref[:, :], preferred_element_type=jnp.float32, precision=lax.Precision.HIGHEST)
    counts = jnp.sum(mask, axis=1, keepdims=True)
    pooled = pooled / jnp.maximum(counts, 1.0)
    hid = jnp.maximum(_dot_def(pooled, w1_ref[:, :]) + b1_ref[:, :], 0.0)
    out_ref[:, :] = _dot_def(hid, w2_ref[:, :]) + b2_ref[:, :]


_pool_cls = pl.pallas_call(
    _pool_body,
    out_shape=jax.ShapeDtypeStruct((NGRAPH, NUM_CLASSES), jnp.float32),
)


def kernel(x, edge_index, edge_attr, batch, node_table, edge_W, edge_b, eps,
           W1, b1, bn1_g, bn1_b, W2, b2, bn2_g, bn2_b,
           cls_W1, cls_b1, cls_W2, cls_b2):
    src = edge_index[0].astype(jnp.int32)
    dst = edge_index[1].astype(jnp.int32)

    h = _embed(x.astype(jnp.int32).reshape(N, 1), node_table)
    ea = _ea_proj(edge_attr, edge_W, edge_b.reshape(1, EMB))
    zeros = jnp.zeros((N, EMB), jnp.float32)

    for i in range(L):
        agg = _sc_msg(h, ea, src, dst, zeros)
        h = _mlp(h, agg, eps[i].reshape(1, 1),
                 W1[i], b1[i].reshape(1, EMB),
                 bn1_g[i].reshape(1, EMB), bn1_b[i].reshape(1, EMB),
                 W2[i], b2[i].reshape(1, EMB),
                 bn2_g[i].reshape(1, EMB), bn2_b[i].reshape(1, EMB))

    return _pool_cls(h, batch.astype(jnp.int32).reshape(1, N),
                     cls_W1, cls_b1.reshape(1, EMB // 2),
                     cls_W2, cls_b2.reshape(1, NUM_CLASSES))


# speedup vs baseline: 3.4034x; 2.0997x over previous
import functools

import jax
import jax.numpy as jnp
from jax import lax
from jax.experimental import pallas as pl
from jax.experimental.pallas import tpu as pltpu
from jax.experimental.pallas import tpu_sc as plsc

N = 10000
E = 320000
EMB = 128
EDGE_DIM = 16
NUM_FEAT = 128
NUM_CLASSES = 6
L = 5
NGRAPH = 64

NCORE = 2
NSUB = 16
NW = NCORE * NSUB
EDGES_PER_W = E // NW
CHUNK = 80
NCHUNK = EDGES_PER_W // CHUNK
ROWS_PER_TILE = 624
ROWS_REM = N - NSUB * ROWS_PER_TILE
VPR = EMB // 16


def _sc_body(h_hbm, ea_hbm, src_hbm, dst_hbm, zeros_hbm, agg_out,
             shared_agg, s0, s1, d0, d1, h0, h1, ea0, ea1,
             isem0, isem1, jsem0, jsem1, gsem0, gsem1,
             esem0, esem1, ssem0, ssem1):
    c = lax.axis_index("c")
    s = lax.axis_index("s")
    wid = c * NSUB + s

    sbufs = (s0, s1)
    dbufs = (d0, d1)
    h_bufs = (h0, h1)
    ea_bufs = (ea0, ea1)
    isems = (isem0, isem1)
    jsems = (jsem0, jsem1)
    gsems = (gsem0, gsem1)
    esems = (esem0, esem1)
    ssems = (ssem0, ssem1)

    pltpu.sync_copy(zeros_hbm.at[pl.ds(s * ROWS_PER_TILE, ROWS_PER_TILE), :],
                    shared_agg.at[pl.ds(s * ROWS_PER_TILE, ROWS_PER_TILE), :])

    @pl.when(s == 0)
    def _zero_rem():
        pltpu.sync_copy(zeros_hbm.at[pl.ds(NSUB * ROWS_PER_TILE, ROWS_REM), :],
                        shared_agg.at[pl.ds(NSUB * ROWS_PER_TILE, ROWS_REM), :])

    plsc.subcore_barrier()

    base_e = wid * EDGES_PER_W

    def issue_src(g, b):
        pltpu.async_copy(src_hbm.at[pl.ds(base_e + g * CHUNK, CHUNK)],
                         sbufs[b], isems[b])

    def issue_dst(g, b):
        pltpu.async_copy(dst_hbm.at[pl.ds(base_e + g * CHUNK, CHUNK)],
                         dbufs[b], jsems[b])

    def issue_gather(b):
        pltpu.async_copy(h_hbm.at[sbufs[b]], h_bufs[b], gsems[b])

    def issue_ea(g, b):
        pltpu.async_copy(ea_hbm.at[pl.ds(base_e + g * CHUNK, CHUNK), :],
                         ea_bufs[b], esems[b])

    def process(g, b, pf):
        nb = 1 - b
        hb = h_bufs[b]
        eb = ea_bufs[b]

        if pf >= 1:
            pltpu.make_async_copy(
                src_hbm.at[pl.ds(base_e, CHUNK)], sbufs[nb], isems[nb]
            ).wait()
            issue_gather(nb)

        pltpu.make_async_copy(h_hbm.at[sbufs[b]], hb, gsems[b]).wait()
        pltpu.make_async_copy(ea_hbm.at[pl.ds(base_e, CHUNK), :],
                              eb, esems[b]).wait()

        if pf == 2:
            issue_src(g + 2, b)

        @pl.when(g > 0)
        def _drain():
            pltpu.make_async_copy(
                ea_bufs[nb], shared_agg.at[dbufs[nb]], ssems[nb]
            ).wait()

        if pf >= 1:
            issue_ea(g + 1, nb)
            issue_dst(g + 1, nb)

        @pl.loop(0, CHUNK)
        def _row(i):
            for j in range(VPR):
                sl = pl.ds(j * 16, 16)
                eb[i, sl] = jnp.maximum(hb[i, sl] + eb[i, sl], 0.0)

        pltpu.make_async_copy(dst_hbm.at[pl.ds(base_e, CHUNK)],
                              dbufs[b], jsems[b]).wait()
        pltpu.async_copy(eb, shared_agg.at[dbufs[b]], ssems[b], add=True)

    pltpu.sync_copy(src_hbm.at[pl.ds(base_e, CHUNK)], sbufs[0])
    issue_gather(0)
    issue_ea(0, 0)
    issue_dst(0, 0)
    issue_src(1, 1)

    @pl.loop(0, NCHUNK - 3, step=2)
    def _pair(g0):
        process(g0, 0, 2)
        process(g0 + 1, 1, 2)

    process(jnp.int32(NCHUNK - 3), 0, 2)
    process(jnp.int32(NCHUNK - 2), 1, 1)
    process(jnp.int32(NCHUNK - 1), 0, 0)

    pltpu.make_async_copy(ea_bufs[0], shared_agg.at[dbufs[0]],
                          ssems[0]).wait()

    plsc.subcore_barrier()
    pltpu.sync_copy(shared_agg.at[pl.ds(s * ROWS_PER_TILE, ROWS_PER_TILE), :],
                    agg_out.at[c, pl.ds(s * ROWS_PER_TILE, ROWS_PER_TILE), :])

    @pl.when(s == 0)
    def _out_rem():
        pltpu.sync_copy(shared_agg.at[pl.ds(NSUB * ROWS_PER_TILE, ROWS_REM), :],
                        agg_out.at[c, pl.ds(NSUB * ROWS_PER_TILE, ROWS_REM), :])


_sc_msg = functools.partial(
    pl.kernel,
    out_type=jax.ShapeDtypeStruct((NCORE, N, EMB), jnp.float32),
    mesh=plsc.VectorSubcoreMesh(core_axis_name="c", subcore_axis_name="s",
                                num_cores=NCORE, num_subcores=NSUB),
    scratch_types=[
        pltpu.VMEM_SHARED((N, EMB), jnp.float32),
        pltpu.VMEM((CHUNK,), jnp.int32),
        pltpu.VMEM((CHUNK,), jnp.int32),
        pltpu.VMEM((CHUNK,), jnp.int32),
        pltpu.VMEM((CHUNK,), jnp.int32),
        pltpu.VMEM((CHUNK, EMB), jnp.float32),
        pltpu.VMEM((CHUNK, EMB), jnp.float32),
        pltpu.VMEM((CHUNK, EMB), jnp.float32),
        pltpu.VMEM((CHUNK, EMB), jnp.float32),
    ] + [pltpu.SemaphoreType.DMA] * 10,
)(_sc_body)


def _dot_def(a, b):
    return jnp.dot(a.astype(jnp.bfloat16), b.astype(jnp.bfloat16),
                   preferred_element_type=jnp.float32)


def _embed_body(x_ref, table_ref, out_ref):
    ids = lax.broadcasted_iota(jnp.int32, (1, NUM_FEAT), 1)
    onehot = (x_ref[:, :] == ids).astype(jnp.float32)
    out_ref[:, :] = jnp.dot(onehot, table_ref[:, :],
                            preferred_element_type=jnp.float32, precision=lax.Precision.HIGHEST)


_embed = pl.pallas_call(
    _embed_body,
    out_shape=jax.ShapeDtypeStruct((N, EMB), jnp.float32),
)


def _ea_body(attr_ref, w_ref, b_ref, out_ref):
    out_ref[:, :] = _dot_def(attr_ref[:, :], w_ref[:, :]) + b_ref[:, :]


_EA_BLK = 4000
_ea_proj = pl.pallas_call(
    _ea_body,
    grid=(E // _EA_BLK,),
    in_specs=[
        pl.BlockSpec((_EA_BLK, EDGE_DIM), lambda i: (i, 0)),
        pl.BlockSpec((EDGE_DIM, EMB), lambda i: (0, 0)),
        pl.BlockSpec((1, EMB), lambda i: (0, 0)),
    ],
    out_specs=pl.BlockSpec((_EA_BLK, EMB), lambda i: (i, 0)),
    out_shape=jax.ShapeDtypeStruct((E, EMB), jnp.float32),
)


def _bn(z, g, b):
    m = jnp.mean(z, axis=0, keepdims=True)
    v = jnp.mean((z - m) ** 2, axis=0, keepdims=True)
    return (z - m) / jnp.sqrt(v + 1e-5) * g + b


def _mlp_body(h_ref, agg_ref, eps_ref, w1_ref, b1_ref, g1_ref, be1_ref,
              w2_ref, b2_ref, g2_ref, be2_ref, out_ref):
    z = (1.0 + eps_ref[0, 0]) * h_ref[:, :] + agg_ref[0] + agg_ref[1]
    z = _dot_def(z, w1_ref[:, :]) + b1_ref[:, :]
    z = jnp.maximum(_bn(z, g1_ref[:, :], be1_ref[:, :]), 0.0)
    z = _dot_def(z, w2_ref[:, :]) + b2_ref[:, :]
    out_ref[:, :] = jnp.maximum(_bn(z, g2_ref[:, :], be2_ref[:, :]), 0.0)


_mlp = pl.pallas_call(
    _mlp_body,
    in_specs=[
        pl.BlockSpec(memory_space=pltpu.VMEM),
        pl.BlockSpec(memory_space=pltpu.VMEM),
        pl.BlockSpec(memory_space=pltpu.SMEM),
        pl.BlockSpec(memory_space=pltpu.VMEM),
        pl.BlockSpec(memory_space=pltpu.VMEM),
        pl.BlockSpec(memory_space=pltpu.VMEM),
        pl.BlockSpec(memory_space=pltpu.VMEM),
        pl.BlockSpec(memory_space=pltpu.VMEM),
        pl.BlockSpec(memory_space=pltpu.VMEM),
        pl.BlockSpec(memory_space=pltpu.VMEM),
        pl.BlockSpec(memory_space=pltpu.VMEM),
    ],
    out_shape=jax.ShapeDtypeStruct((N, EMB), jnp.float32),
)


def _pool_body(h_ref, batch_ref, w1_ref, b1_ref, w2_ref, b2_ref, out_ref):
    gids = lax.broadcasted_iota(jnp.int32, (NGRAPH, 1), 0)
    mask = (batch_ref[:, :] == gids).astype(jnp.float32)
    pooled = jnp.dot(mask, h_ref[:, :], preferred_element_type=jnp.float32, precision=lax.Precision.HIGHEST)
    counts = jnp.sum(mask, axis=1, keepdims=True)
    pooled = pooled / jnp.maximum(counts, 1.0)
    hid = jnp.maximum(_dot_def(pooled, w1_ref[:, :]) + b1_ref[:, :], 0.0)
    out_ref[:, :] = _dot_def(hid, w2_ref[:, :]) + b2_ref[:, :]


_pool_cls = pl.pallas_call(
    _pool_body,
    out_shape=jax.ShapeDtypeStruct((NGRAPH, NUM_CLASSES), jnp.float32),
)


def kernel(x, edge_index, edge_attr, batch, node_table, edge_W, edge_b, eps,
           W1, b1, bn1_g, bn1_b, W2, b2, bn2_g, bn2_b,
           cls_W1, cls_b1, cls_W2, cls_b2):
    src = edge_index[0].astype(jnp.int32)
    dst = edge_index[1].astype(jnp.int32)

    h = _embed(x.astype(jnp.int32).reshape(N, 1), node_table)
    ea = _ea_proj(edge_attr, edge_W, edge_b.reshape(1, EMB))
    zeros = jnp.zeros((N, EMB), jnp.float32)

    for i in range(L):
        agg = _sc_msg(h, ea, src, dst, zeros)
        h = _mlp(h, agg, eps[i].reshape(1, 1),
                 W1[i], b1[i].reshape(1, EMB),
                 bn1_g[i].reshape(1, EMB), bn1_b[i].reshape(1, EMB),
                 W2[i], b2[i].reshape(1, EMB),
                 bn2_g[i].reshape(1, EMB), bn2_b[i].reshape(1, EMB))

    return _pool_cls(h, batch.astype(jnp.int32).reshape(1, N),
                     cls_W1, cls_b1.reshape(1, EMB // 2),
                     cls_W2, cls_b2.reshape(1, NUM_CLASSES))
